# Initial kernel scaffold; baseline (speedup 1.0000x reference)
import jax
import jax.numpy as jnp
from jax.experimental import pallas as pl


def kernel(condition, tables, W1, b1, W2, b2):
    raise NotImplementedError("write your pallas kernel here")


# speedup vs baseline: 2.1070x; 2.1070x over previous
import functools

import jax
import jax.numpy as jnp
from jax import lax
from jax.experimental import pallas as pl
from jax.experimental.pallas import tpu as pltpu
from jax.experimental.pallas import tpu_sc as plsc

NUM_KEYS = 8
CARDINALITY = 100000
PER_KEY_DIM = 16
COND_DIM = 128
BATCH = 16384

ROWS = BATCH * NUM_KEYS

NC, NS = 2, 16
NW = NC * NS
ROWS_PER_W = ROWS // NW

@functools.lru_cache(maxsize=None)
def _make_gather_rows():
    mesh = plsc.VectorSubcoreMesh(core_axis_name="c", subcore_axis_name="s")

    @functools.partial(
        pl.kernel,
        mesh=mesh,
        out_type=jax.ShapeDtypeStruct((ROWS, PER_KEY_DIM), jnp.float32),
        scratch_types=[
            pltpu.VMEM((ROWS_PER_W,), jnp.int32),
            pltpu.VMEM((ROWS_PER_W, PER_KEY_DIM), jnp.float32),
            pltpu.SemaphoreType.DMA,
        ],
        compiler_params=pltpu.CompilerParams(use_tc_tiling_on_sc=False),
    )
    def _gather_rows(idx_hbm, table_hbm, out_hbm, idx_v, rows_v, sem):
        wid = lax.axis_index("s") * NC + lax.axis_index("c")
        base = wid * ROWS_PER_W
        pltpu.sync_copy(idx_hbm.at[pl.ds(base, ROWS_PER_W)], idx_v)
        pltpu.async_copy(table_hbm.at[idx_v], rows_v, sem).wait()
        pltpu.sync_copy(rows_v, out_hbm.at[pl.ds(base, ROWS_PER_W)])

    return _gather_rows


def _mlp_body(x_ref, w1_ref, b1_ref, w2_ref, b2_ref, o_ref):
    x = x_ref[...]
    h = lax.dot_general(x, w1_ref[...], (((1,), (1,)), ((), ())),
                        preferred_element_type=jnp.float32)
    h = jnp.maximum(h + b1_ref[...], 0.0)
    o = lax.dot_general(h, w2_ref[...], (((1,), (1,)), ((), ())),
                        preferred_element_type=jnp.float32)
    o_ref[...] = o + b2_ref[...]


def _mlp(x, W1, b1, W2, b2, block=2048):
    B = x.shape[0]
    return pl.pallas_call(
        _mlp_body,
        grid=(B // block,),
        in_specs=[
            pl.BlockSpec((block, COND_DIM), lambda i: (i, 0)),
            pl.BlockSpec((COND_DIM, COND_DIM), lambda i: (0, 0)),
            pl.BlockSpec((1, COND_DIM), lambda i: (0, 0)),
            pl.BlockSpec((COND_DIM, COND_DIM), lambda i: (0, 0)),
            pl.BlockSpec((1, COND_DIM), lambda i: (0, 0)),
        ],
        out_specs=pl.BlockSpec((block, COND_DIM), lambda i: (i, 0)),
        out_shape=jax.ShapeDtypeStruct((B, COND_DIM), jnp.float32),
    )(x, W1, b1.reshape(1, COND_DIM), W2, b2.reshape(1, COND_DIM))


def kernel(condition, tables, W1, b1, W2, b2):
    offsets = jnp.arange(NUM_KEYS, dtype=jnp.int32) * CARDINALITY
    flat_idx = (condition.astype(jnp.int32) + offsets).reshape(ROWS)
    table = tables.reshape(NUM_KEYS * CARDINALITY, PER_KEY_DIM)
    h = _make_gather_rows()(flat_idx, table)
    return _mlp(h.reshape(BATCH, COND_DIM), W1, b1, W2, b2)


# speedup vs baseline: 2.9033x; 1.3779x over previous
import functools

import jax
import jax.numpy as jnp
from jax import lax
from jax.experimental import pallas as pl
from jax.experimental.pallas import tpu as pltpu
from jax.experimental.pallas import tpu_sc as plsc

NUM_KEYS = 8
CARDINALITY = 100000
PER_KEY_DIM = 16
COND_DIM = 128
BATCH = 16384

ROWS = BATCH * NUM_KEYS

NC, NS = 2, 16
NW = NC * NS
ROWS_PER_W = ROWS // NW

PACK_CBLK = 1024
PACK_CPAD = 100352


def _pack_body(x_ref, o_ref):
    for k in range(NUM_KEYS):
        o_ref[:, k * PER_KEY_DIM:(k + 1) * PER_KEY_DIM] = x_ref[k].T


def _pack(t2):
    return pl.pallas_call(
        _pack_body,
        grid=(PACK_CPAD // PACK_CBLK,),
        in_specs=[
            pl.BlockSpec((NUM_KEYS, PER_KEY_DIM, PACK_CBLK), lambda j: (0, 0, j)),
        ],
        out_specs=pl.BlockSpec((PACK_CBLK, COND_DIM), lambda j: (j, 0)),
        out_shape=jax.ShapeDtypeStruct((PACK_CPAD, COND_DIM), jnp.float32),
    )(t2)


@functools.lru_cache(maxsize=None)
def _make_gather_rows():
    mesh = plsc.VectorSubcoreMesh(core_axis_name="c", subcore_axis_name="s")

    @functools.partial(
        pl.kernel,
        mesh=mesh,
        out_type=jax.ShapeDtypeStruct((ROWS, PER_KEY_DIM), jnp.float32),
        scratch_types=[
            pltpu.VMEM((ROWS_PER_W,), jnp.int32),
            pltpu.VMEM((ROWS_PER_W, PER_KEY_DIM), jnp.float32),
            pltpu.SemaphoreType.DMA,
        ],
        compiler_params=pltpu.CompilerParams(use_tc_tiling_on_sc=False),
    )
    def _gather_rows(idx_hbm, table_hbm, out_hbm, idx_v, rows_v, sem):
        wid = lax.axis_index("s") * NC + lax.axis_index("c")
        base = wid * ROWS_PER_W
        pltpu.sync_copy(idx_hbm.at[pl.ds(base, ROWS_PER_W)], idx_v)
        pltpu.async_copy(table_hbm.at[idx_v], rows_v, sem).wait()
        pltpu.sync_copy(rows_v, out_hbm.at[pl.ds(base, ROWS_PER_W)])

    return _gather_rows


def _mlp_body(x_ref, w1_ref, b1_ref, w2_ref, b2_ref, o_ref):
    x = x_ref[...]
    h = lax.dot_general(x, w1_ref[...], (((1,), (1,)), ((), ())),
                        preferred_element_type=jnp.float32)
    h = jnp.maximum(h + b1_ref[...], 0.0)
    o = lax.dot_general(h, w2_ref[...], (((1,), (1,)), ((), ())),
                        preferred_element_type=jnp.float32)
    o_ref[...] = o + b2_ref[...]


def _mlp(x, W1, b1, W2, b2, block=2048):
    B = x.shape[0]
    return pl.pallas_call(
        _mlp_body,
        grid=(B // block,),
        in_specs=[
            pl.BlockSpec((block, COND_DIM), lambda i: (i, 0)),
            pl.BlockSpec((COND_DIM, COND_DIM), lambda i: (0, 0)),
            pl.BlockSpec((1, COND_DIM), lambda i: (0, 0)),
            pl.BlockSpec((COND_DIM, COND_DIM), lambda i: (0, 0)),
            pl.BlockSpec((1, COND_DIM), lambda i: (0, 0)),
        ],
        out_specs=pl.BlockSpec((block, COND_DIM), lambda i: (i, 0)),
        out_shape=jax.ShapeDtypeStruct((B, COND_DIM), jnp.float32),
    )(x, W1, b1.reshape(1, COND_DIM), W2, b2.reshape(1, COND_DIM))


def kernel(condition, tables, W1, b1, W2, b2):
    t2 = jnp.swapaxes(tables, 1, 2)
    fused = _pack(t2)
    table = fused.reshape(PACK_CPAD * NUM_KEYS, PER_KEY_DIM)
    offsets = jnp.arange(NUM_KEYS, dtype=jnp.int32)
    flat_idx = (condition.astype(jnp.int32) * NUM_KEYS + offsets).reshape(ROWS)
    h = _make_gather_rows()(flat_idx, table)
    return _mlp(h.reshape(BATCH, COND_DIM), W1, b1, W2, b2)


# speedup vs baseline: 5.8764x; 2.0240x over previous
import functools

import jax
import jax.numpy as jnp
from jax import lax
from jax.experimental import pallas as pl
from jax.experimental.pallas import tpu as pltpu
from jax.experimental.pallas import tpu_sc as plsc

NUM_KEYS = 8
CARDINALITY = 100000
PER_KEY_DIM = 16
COND_DIM = 128
BATCH = 16384

ROWS = BATCH * NUM_KEYS

NC, NS = 2, 16
NW = NC * NS
ROWS_PER_W = ROWS // NW

PACK_CBLK = 1024
PACK_CPAD = 100352


def _pack_body(x_ref, o_ref):
    o_ref[...] = x_ref[...].reshape(COND_DIM, PACK_CBLK).T


def _pack(t2):
    return pl.pallas_call(
        _pack_body,
        grid=(PACK_CPAD // PACK_CBLK,),
        in_specs=[
            pl.BlockSpec((NUM_KEYS, PER_KEY_DIM, PACK_CBLK), lambda j: (0, 0, j)),
        ],
        out_specs=pl.BlockSpec((PACK_CBLK, COND_DIM), lambda j: (j, 0)),
        out_shape=jax.ShapeDtypeStruct((PACK_CPAD, COND_DIM), jnp.float32),
    )(t2)


@functools.lru_cache(maxsize=None)
def _make_gather_rows():
    mesh = plsc.VectorSubcoreMesh(core_axis_name="c", subcore_axis_name="s")

    @functools.partial(
        pl.kernel,
        mesh=mesh,
        out_type=jax.ShapeDtypeStruct((ROWS, PER_KEY_DIM), jnp.float32),
        scratch_types=[
            pltpu.VMEM((ROWS_PER_W,), jnp.int32),
            pltpu.VMEM((ROWS_PER_W, PER_KEY_DIM), jnp.float32),
            pltpu.SemaphoreType.DMA,
        ],
        compiler_params=pltpu.CompilerParams(use_tc_tiling_on_sc=False),
    )
    def _gather_rows(idx_hbm, table_hbm, out_hbm, idx_v, rows_v, sem):
        wid = lax.axis_index("s") * NC + lax.axis_index("c")
        base = wid * ROWS_PER_W
        pltpu.sync_copy(idx_hbm.at[pl.ds(base, ROWS_PER_W)], idx_v)
        pltpu.async_copy(table_hbm.at[idx_v], rows_v, sem).wait()
        pltpu.sync_copy(rows_v, out_hbm.at[pl.ds(base, ROWS_PER_W)])

    return _gather_rows


def _mlp_body(x_ref, w1_ref, b1_ref, w2_ref, b2_ref, o_ref):
    x = x_ref[...]
    h = lax.dot_general(x, w1_ref[...], (((1,), (1,)), ((), ())),
                        preferred_element_type=jnp.float32)
    h = jnp.maximum(h + b1_ref[...], 0.0)
    o = lax.dot_general(h, w2_ref[...], (((1,), (1,)), ((), ())),
                        preferred_element_type=jnp.float32)
    o_ref[...] = o + b2_ref[...]


def _mlp(x, W1, b1, W2, b2, block=2048):
    B = x.shape[0]
    return pl.pallas_call(
        _mlp_body,
        grid=(B // block,),
        in_specs=[
            pl.BlockSpec((block, COND_DIM), lambda i: (i, 0)),
            pl.BlockSpec((COND_DIM, COND_DIM), lambda i: (0, 0)),
            pl.BlockSpec((1, COND_DIM), lambda i: (0, 0)),
            pl.BlockSpec((COND_DIM, COND_DIM), lambda i: (0, 0)),
            pl.BlockSpec((1, COND_DIM), lambda i: (0, 0)),
        ],
        out_specs=pl.BlockSpec((block, COND_DIM), lambda i: (i, 0)),
        out_shape=jax.ShapeDtypeStruct((B, COND_DIM), jnp.float32),
    )(x, W1, b1.reshape(1, COND_DIM), W2, b2.reshape(1, COND_DIM))


def kernel(condition, tables, W1, b1, W2, b2):
    t2 = jnp.swapaxes(tables, 1, 2)
    fused = _pack(t2)
    table = fused.reshape(PACK_CPAD * NUM_KEYS, PER_KEY_DIM)
    offsets = jnp.arange(NUM_KEYS, dtype=jnp.int32)
    flat_idx = (condition.astype(jnp.int32) * NUM_KEYS + offsets).reshape(ROWS)
    h = _make_gather_rows()(flat_idx, table)
    return _mlp(h.reshape(BATCH, COND_DIM), W1, b1, W2, b2)


# speedup vs baseline: 9.1334x; 1.5542x over previous
import functools

import jax
import jax.numpy as jnp
from jax import lax
from jax.experimental import pallas as pl
from jax.experimental.pallas import tpu as pltpu
from jax.experimental.pallas import tpu_sc as plsc

NUM_KEYS = 8
CARDINALITY = 100000
PER_KEY_DIM = 16
COND_DIM = 128
BATCH = 16384

ROWS = BATCH * NUM_KEYS

NC, NS = 2, 16
NW = NC * NS
ROWS_PER_W = ROWS // NW

PACK_CBLK = 4096
PACK_CPAD = 102400


def _pack_body(x_ref, o_ref):
    o_ref[...] = x_ref[...].reshape(COND_DIM, PACK_CBLK).T


def _pack(t2):
    return pl.pallas_call(
        _pack_body,
        grid=(PACK_CPAD // PACK_CBLK,),
        in_specs=[
            pl.BlockSpec((NUM_KEYS, PER_KEY_DIM, PACK_CBLK), lambda j: (0, 0, j)),
        ],
        out_specs=pl.BlockSpec((PACK_CBLK, COND_DIM), lambda j: (j, 0)),
        out_shape=jax.ShapeDtypeStruct((PACK_CPAD, COND_DIM), jnp.float32),
    )(t2)


B_PER_W = BATCH // (NW // NUM_KEYS)


@functools.lru_cache(maxsize=None)
def _make_gather_rows():
    mesh = plsc.VectorSubcoreMesh(core_axis_name="c", subcore_axis_name="s")

    @functools.partial(
        pl.kernel,
        mesh=mesh,
        out_type=jax.ShapeDtypeStruct((BATCH, COND_DIM), jnp.float32),
        scratch_types=[
            pltpu.VMEM((B_PER_W,), jnp.int32),
            pltpu.VMEM((B_PER_W,), jnp.int32),
            pltpu.VMEM((B_PER_W, PER_KEY_DIM), jnp.float32),
            pltpu.SemaphoreType.DMA,
        ],
        compiler_params=pltpu.CompilerParams(use_tc_tiling_on_sc=False),
    )
    def _gather_rows(cond_hbm, table_hbm, out_hbm, c_v, idx_v, rows_v, sem):
        wid = lax.axis_index("s") * NC + lax.axis_index("c")
        k = wid % NUM_KEYS
        b0 = (wid // NUM_KEYS) * B_PER_W
        pltpu.sync_copy(cond_hbm.at[k, pl.ds(b0, B_PER_W)], c_v)

        def body(i, carry):
            idx_v[pl.ds(i * 16, 16)] = c_v[pl.ds(i * 16, 16)] * NUM_KEYS + k
            return carry

        lax.fori_loop(0, B_PER_W // 16, body, 0)
        pltpu.async_copy(table_hbm.at[idx_v], rows_v, sem).wait()
        pltpu.sync_copy(
            rows_v,
            out_hbm.at[pl.ds(b0, B_PER_W),
                       pl.ds(k * PER_KEY_DIM, PER_KEY_DIM)],
        )

    return _gather_rows


def _mlp_body(x_ref, w1_ref, b1_ref, w2_ref, b2_ref, o_ref):
    x = x_ref[...]
    h = lax.dot_general(x, w1_ref[...], (((1,), (1,)), ((), ())),
                        preferred_element_type=jnp.float32)
    h = jnp.maximum(h + b1_ref[...], 0.0)
    o = lax.dot_general(h, w2_ref[...], (((1,), (1,)), ((), ())),
                        preferred_element_type=jnp.float32)
    o_ref[...] = o + b2_ref[...]


def _mlp(x, W1, b1, W2, b2, block=2048):
    B = x.shape[0]
    return pl.pallas_call(
        _mlp_body,
        grid=(B // block,),
        in_specs=[
            pl.BlockSpec((block, COND_DIM), lambda i: (i, 0)),
            pl.BlockSpec((COND_DIM, COND_DIM), lambda i: (0, 0)),
            pl.BlockSpec((1, COND_DIM), lambda i: (0, 0)),
            pl.BlockSpec((COND_DIM, COND_DIM), lambda i: (0, 0)),
            pl.BlockSpec((1, COND_DIM), lambda i: (0, 0)),
        ],
        out_specs=pl.BlockSpec((block, COND_DIM), lambda i: (i, 0)),
        out_shape=jax.ShapeDtypeStruct((B, COND_DIM), jnp.float32),
    )(x, W1, b1.reshape(1, COND_DIM), W2, b2.reshape(1, COND_DIM))


def kernel(condition, tables, W1, b1, W2, b2):
    t2 = jnp.swapaxes(tables, 1, 2)
    fused = _pack(t2)
    table = fused.reshape(PACK_CPAD * NUM_KEYS, PER_KEY_DIM)
    cond_t = condition.T.astype(jnp.int32)
    h = _make_gather_rows()(cond_t, table)
    return _mlp(h, W1, b1, W2, b2)


# speedup vs baseline: 10.0609x; 1.1016x over previous
import functools

import jax
import jax.numpy as jnp
from jax import lax
from jax.experimental import pallas as pl
from jax.experimental.pallas import tpu as pltpu
from jax.experimental.pallas import tpu_sc as plsc

NUM_KEYS = 8
CARDINALITY = 100000
PER_KEY_DIM = 16
COND_DIM = 128
BATCH = 16384

ROWS = BATCH * NUM_KEYS

NC, NS = 2, 16
NW = NC * NS
ROWS_PER_W = ROWS // NW

PACK_CBLK = 8192
PACK_CPAD = 106496


def _pack_body(x_ref, o_ref):
    o_ref[...] = x_ref[...].reshape(COND_DIM, PACK_CBLK).T


def _pack(t2):
    return pl.pallas_call(
        _pack_body,
        grid=(PACK_CPAD // PACK_CBLK,),
        in_specs=[
            pl.BlockSpec((NUM_KEYS, PER_KEY_DIM, PACK_CBLK), lambda j: (0, 0, j)),
        ],
        out_specs=pl.BlockSpec((PACK_CBLK, COND_DIM), lambda j: (j, 0)),
        out_shape=jax.ShapeDtypeStruct((PACK_CPAD, COND_DIM), jnp.float32),
    )(t2)


B_PER_W = BATCH // (NW // NUM_KEYS)


@functools.lru_cache(maxsize=None)
def _make_gather_rows():
    mesh = plsc.VectorSubcoreMesh(core_axis_name="c", subcore_axis_name="s")

    @functools.partial(
        pl.kernel,
        mesh=mesh,
        out_type=jax.ShapeDtypeStruct((BATCH, COND_DIM), jnp.float32),
        scratch_types=[
            pltpu.VMEM((B_PER_W,), jnp.int32),
            pltpu.VMEM((B_PER_W,), jnp.int32),
            pltpu.VMEM((B_PER_W, PER_KEY_DIM), jnp.float32),
            pltpu.SemaphoreType.DMA,
        ],
        compiler_params=pltpu.CompilerParams(use_tc_tiling_on_sc=False),
    )
    def _gather_rows(cond_hbm, table_hbm, out_hbm, c_v, idx_v, rows_v, sem):
        wid = lax.axis_index("s") * NC + lax.axis_index("c")
        k = wid % NUM_KEYS
        b0 = (wid // NUM_KEYS) * B_PER_W
        pltpu.sync_copy(cond_hbm.at[k, pl.ds(b0, B_PER_W)], c_v)

        def body(i, carry):
            idx_v[pl.ds(i * 16, 16)] = c_v[pl.ds(i * 16, 16)] * NUM_KEYS + k
            return carry

        lax.fori_loop(0, B_PER_W // 16, body, 0)
        pltpu.async_copy(table_hbm.at[idx_v], rows_v, sem).wait()
        pltpu.sync_copy(
            rows_v,
            out_hbm.at[pl.ds(b0, B_PER_W),
                       pl.ds(k * PER_KEY_DIM, PER_KEY_DIM)],
        )

    return _gather_rows


def _mlp_body(x_ref, w1_ref, b1_ref, w2_ref, b2_ref, o_ref):
    x = x_ref[...]
    h = lax.dot_general(x, w1_ref[...], (((1,), (1,)), ((), ())),
                        preferred_element_type=jnp.float32)
    h = jnp.maximum(h + b1_ref[...], 0.0)
    o = lax.dot_general(h, w2_ref[...], (((1,), (1,)), ((), ())),
                        preferred_element_type=jnp.float32)
    o_ref[...] = o + b2_ref[...]


def _mlp(x, W1, b1, W2, b2, block=4096):
    B = x.shape[0]
    return pl.pallas_call(
        _mlp_body,
        grid=(B // block,),
        in_specs=[
            pl.BlockSpec((block, COND_DIM), lambda i: (i, 0)),
            pl.BlockSpec((COND_DIM, COND_DIM), lambda i: (0, 0)),
            pl.BlockSpec((1, COND_DIM), lambda i: (0, 0)),
            pl.BlockSpec((COND_DIM, COND_DIM), lambda i: (0, 0)),
            pl.BlockSpec((1, COND_DIM), lambda i: (0, 0)),
        ],
        out_specs=pl.BlockSpec((block, COND_DIM), lambda i: (i, 0)),
        out_shape=jax.ShapeDtypeStruct((B, COND_DIM), jnp.float32),
    )(x, W1, b1.reshape(1, COND_DIM), W2, b2.reshape(1, COND_DIM))


def kernel(condition, tables, W1, b1, W2, b2):
    t2 = jnp.swapaxes(tables, 1, 2)
    fused = _pack(t2)
    table = fused.reshape(PACK_CPAD * NUM_KEYS, PER_KEY_DIM)
    cond_t = condition.T.astype(jnp.int32)
    h = _make_gather_rows()(cond_t, table)
    return _mlp(h, W1, b1, W2, b2)
